# baseline (speedup 1.0000x reference)
import jax
import jax.numpy as jnp
from jax import lax
from jax.experimental import pallas as pl
from jax.experimental.pallas import tpu as pltpu
from jax.experimental.pallas import tpu_sc as plsc

VOCAB_SIZE = 1000000
EMBED = 64
FEAT = 512
BATCH_N = 16384
HIST_N = 200

NUM_CORES = 2
NUM_SUBCORES = 16
NUM_WORKERS = NUM_CORES * NUM_SUBCORES
B_PER_W = BATCH_N // NUM_WORKERS
SG = 64
NGROUPS = B_PER_W // SG
NBUF = 8
IDX_CHUNK0 = 128
IDX_CHUNK1 = HIST_N - IDX_CHUNK0


def _sc_gather_pool_body(idx_hbm, table_hbm, out_hbm,
                         idx_v, r0, r1, r2, r3, r4, r5, r6, r7, out_v,
                         m0, m1, m2, m3, m4, m5, m6, m7):
  rows = (r0, r1, r2, r3, r4, r5, r6, r7)
  sems = (m0, m1, m2, m3, m4, m5, m6, m7)
  table2d = table_hbm
  c = lax.axis_index("c")
  s = lax.axis_index("s")
  wid = s * NUM_CORES + c
  base = wid * B_PER_W

  def start_gather(samp, rows_ref, sem):
    off = samp * HIST_N
    pltpu.async_copy(table2d.at[idx_v.at[pl.ds(off, IDX_CHUNK0)]],
                     rows_ref.at[pl.ds(0, IDX_CHUNK0)], sem)
    pltpu.async_copy(table2d.at[idx_v.at[pl.ds(off + IDX_CHUNK0, IDX_CHUNK1)]],
                     rows_ref.at[pl.ds(IDX_CHUNK0, IDX_CHUNK1)], sem)

  def wait_gather(rows_ref, sem):
    pltpu.make_async_copy(table2d.at[pl.ds(0, HIST_N)], rows_ref, sem).wait()

  def reduce_rows(rows_ref, samp):
    def h_body(h, accs):
      return tuple(accs[d] + rows_ref[h, pl.ds(d * 16, 16)] for d in range(4))
    accs = tuple(jnp.zeros((16,), jnp.float32) for _ in range(4))
    accs = lax.fori_loop(0, HIST_N, h_body, accs, unroll=8)
    for d in range(4):
      out_v[samp, pl.ds(d * 16, 16)] = accs[d]

  def g_body(g, carry):
    gbase = (base + g * SG) * HIST_N
    pltpu.sync_copy(idx_hbm.at[pl.ds(gbase, SG * HIST_N)], idx_v)
    for j in range(NBUF):
      start_gather(j, rows[j], sems[j])

    def p_body(p, carry2):
      for j in range(NBUF):
        samp = NBUF * p + j
        wait_gather(rows[j], sems[j])
        reduce_rows(rows[j], samp)
        start_gather(jnp.minimum(samp + NBUF, SG - 1), rows[j], sems[j])
      return carry2

    lax.fori_loop(0, SG // NBUF, p_body, 0)
    for j in range(NBUF):
      wait_gather(rows[j], sems[j])
    pltpu.sync_copy(out_v, out_hbm.at[pl.ds(base + g * SG, SG)])
    return carry

  lax.fori_loop(0, NGROUPS, g_body, 0)


@jax.jit
def _sc_gather_pool(idx_flat, table):
  mesh = plsc.VectorSubcoreMesh(core_axis_name="c", subcore_axis_name="s")
  f = pl.kernel(
      _sc_gather_pool_body,
      out_type=jax.ShapeDtypeStruct((BATCH_N, EMBED), jnp.float32),
      mesh=mesh,
      scratch_types=(
          [pltpu.VMEM((SG * HIST_N,), jnp.int32)]
          + [pltpu.VMEM((HIST_N, EMBED), jnp.float32) for _ in range(NBUF)]
          + [pltpu.VMEM((SG, EMBED), jnp.float32)]
          + [pltpu.SemaphoreType.DMA for _ in range(NBUF)]
      ),
      compiler_params=pltpu.CompilerParams(use_tc_tiling_on_sc=False),
  )
  return f(idx_flat, table)


def _mm_body(x_ref, w_ref, b_ref, o_ref):
  o_ref[...] = (
      jnp.dot(x_ref[...], w_ref[...], preferred_element_type=jnp.float32)
      * (1.0 / VOCAB_SIZE)
      + b_ref[...]
  )


@jax.jit
def _project(sums, W, b):
  bm = 2048
  return pl.pallas_call(
      _mm_body,
      grid=(BATCH_N // bm,),
      in_specs=[
          pl.BlockSpec((bm, EMBED), lambda i: (i, 0)),
          pl.BlockSpec((EMBED, FEAT), lambda i: (0, 0)),
          pl.BlockSpec((1, FEAT), lambda i: (0, 0)),
      ],
      out_specs=pl.BlockSpec((bm, FEAT), lambda i: (i, 0)),
      out_shape=jax.ShapeDtypeStruct((BATCH_N, FEAT), jnp.float32),
  )(sums, W, b.reshape(1, FEAT))


def kernel(text_input, table, W, b):
  idx = text_input.reshape(-1)
  if idx.dtype != jnp.int32:
    idx = idx.astype(jnp.int32)
  sums = _sc_gather_pool(idx, table)
  return _project(sums, W, b)


# speedup vs baseline: 1.0378x; 1.0378x over previous
import jax
import jax.numpy as jnp
from jax import lax
from jax.experimental import pallas as pl
from jax.experimental.pallas import tpu as pltpu
from jax.experimental.pallas import tpu_sc as plsc

VOCAB_SIZE = 1000000
EMBED = 64
FEAT = 512
BATCH_N = 16384
HIST_N = 200

NUM_CORES = 2
NUM_SUBCORES = 16
NUM_WORKERS = NUM_CORES * NUM_SUBCORES
B_PER_W = BATCH_N // NUM_WORKERS
SG = 128
NGROUPS = B_PER_W // SG
NBUF = 4
IDX_CHUNK0 = 128
IDX_CHUNK1 = HIST_N - IDX_CHUNK0


def _sc_gather_pool_body(idx_hbm, table_hbm, out_hbm,
                         idx_v, r0, r1, r2, r3, out_v, m0, m1, m2, m3):
  rows = (r0, r1, r2, r3)
  sems = (m0, m1, m2, m3)
  table2d = table_hbm
  c = lax.axis_index("c")
  s = lax.axis_index("s")
  wid = s * NUM_CORES + c
  base = wid * B_PER_W

  def start_gather(samp, rows_ref, sem):
    off = samp * HIST_N
    pltpu.async_copy(table2d.at[idx_v.at[pl.ds(off, IDX_CHUNK0)]],
                     rows_ref.at[pl.ds(0, IDX_CHUNK0)], sem)
    pltpu.async_copy(table2d.at[idx_v.at[pl.ds(off + IDX_CHUNK0, IDX_CHUNK1)]],
                     rows_ref.at[pl.ds(IDX_CHUNK0, IDX_CHUNK1)], sem)

  def wait_gather(rows_ref, sem):
    pltpu.make_async_copy(table2d.at[pl.ds(0, HIST_N)], rows_ref, sem).wait()

  def reduce_rows(rows_ref, samp):
    def h_body(h, accs):
      return tuple(accs[d] + rows_ref[h, pl.ds(d * 16, 16)] for d in range(4))
    accs = tuple(jnp.zeros((16,), jnp.float32) for _ in range(4))
    accs = lax.fori_loop(0, HIST_N, h_body, accs, unroll=8)
    for d in range(4):
      out_v[samp, pl.ds(d * 16, 16)] = accs[d]

  def g_body(g, carry):
    gbase = (base + g * SG) * HIST_N
    pltpu.sync_copy(idx_hbm.at[pl.ds(gbase, SG * HIST_N)], idx_v)
    for j in range(NBUF):
      start_gather(j, rows[j], sems[j])

    def p_body(p, carry2):
      for j in range(NBUF):
        samp = NBUF * p + j
        wait_gather(rows[j], sems[j])
        reduce_rows(rows[j], samp)
        start_gather(jnp.minimum(samp + NBUF, SG - 1), rows[j], sems[j])
      return carry2

    lax.fori_loop(0, SG // NBUF, p_body, 0)
    for j in range(NBUF):
      wait_gather(rows[j], sems[j])
    pltpu.sync_copy(out_v, out_hbm.at[pl.ds(base + g * SG, SG)])
    return carry

  lax.fori_loop(0, NGROUPS, g_body, 0)


@jax.jit
def _sc_gather_pool(idx_flat, table):
  mesh = plsc.VectorSubcoreMesh(core_axis_name="c", subcore_axis_name="s")
  f = pl.kernel(
      _sc_gather_pool_body,
      out_type=jax.ShapeDtypeStruct((BATCH_N, EMBED), jnp.float32),
      mesh=mesh,
      scratch_types=(
          [pltpu.VMEM((SG * HIST_N,), jnp.int32)]
          + [pltpu.VMEM((HIST_N, EMBED), jnp.float32) for _ in range(NBUF)]
          + [pltpu.VMEM((SG, EMBED), jnp.float32)]
          + [pltpu.SemaphoreType.DMA for _ in range(NBUF)]
      ),
      compiler_params=pltpu.CompilerParams(use_tc_tiling_on_sc=False),
  )
  return f(idx_flat, table)


def _mm_body(x_ref, w_ref, b_ref, o_ref):
  o_ref[...] = (
      jnp.dot(x_ref[...], w_ref[...], preferred_element_type=jnp.float32)
      * (1.0 / VOCAB_SIZE)
      + b_ref[...]
  )


@jax.jit
def _project(sums, W, b):
  bm = 2048
  return pl.pallas_call(
      _mm_body,
      grid=(BATCH_N // bm,),
      in_specs=[
          pl.BlockSpec((bm, EMBED), lambda i: (i, 0)),
          pl.BlockSpec((EMBED, FEAT), lambda i: (0, 0)),
          pl.BlockSpec((1, FEAT), lambda i: (0, 0)),
      ],
      out_specs=pl.BlockSpec((bm, FEAT), lambda i: (i, 0)),
      out_shape=jax.ShapeDtypeStruct((BATCH_N, FEAT), jnp.float32),
  )(sums, W, b.reshape(1, FEAT))


def kernel(text_input, table, W, b):
  idx = text_input.reshape(-1)
  if idx.dtype != jnp.int32:
    idx = idx.astype(jnp.int32)
  sums = _sc_gather_pool(idx, table)
  return _project(sums, W, b)
